# baseline (speedup 1.0000x reference)
import jax
import jax.numpy as jnp
from jax import lax
from jax.experimental import pallas as pl
from jax.experimental.pallas import tpu as pltpu
from jax.experimental.pallas import tpu_sc as plsc

B = 1024
L = 20
ND = 26
NF = ND + 1
NC_FEAT = 22
NC_PAD = 32
V = 100000
D = 64
BL = B * L

NWORKERS = 32
PAIRS_PER_W = BL // NWORKERS
K = 32
NCHUNK = PAIRS_PER_W // K
IDX_PER_CHUNK = K * NF
G_SUB = 4
SUB = G_SUB * NF
NSUB = IDX_PER_CHUNK // SUB
IDX_ROWS = BL * NF // SUB
ROWS_PER_W = IDX_ROWS // NWORKERS
D_SL = D // 16


def _sc_body(table_hbm, idx_hbm, x_hbm, w_hbm, b_hbm, out_hbm,
             idx_v, out_v, x_v, w_v, b_v, sem):
    wid = lax.axis_index("s") * 2 + lax.axis_index("c")

    pltpu.sync_copy(x_hbm.at[pl.ds(wid * PAIRS_PER_W * NC_PAD,
                                   PAIRS_PER_W * NC_PAD)], x_v)
    pltpu.sync_copy(w_hbm, w_v)
    pltpu.sync_copy(b_hbm, b_v)

    def chunk_body(ci, carry):
        idx_row0 = wid * ROWS_PER_W + ci * NSUB
        out_row0 = (wid * PAIRS_PER_W + ci * K) * NF

        pltpu.sync_copy(idx_hbm.at[pl.ds(idx_row0, NSUB)], idx_v)

        copies = []
        for j in range(NSUB):
            copies.append(pltpu.async_copy(
                table_hbm.at[idx_v.at[j]],
                out_v.at[pl.ds(j * SUB, SUB)],
                sem))
        for cp in copies:
            cp.wait()

        def pair_body(k, c2):
            row = k * NF + (NF - 1)
            xbase = pl.multiple_of((ci * K + k) * NC_PAD, NC_PAD)
            v0 = x_v[pl.ds(xbase, 16)]
            v1 = x_v[pl.ds(xbase + 16, 16)]
            accs = [b_v[pl.ds(s * 16, 16)] for s in range(D_SL)]
            for c in range(NC_FEAT):
                xs = v0[c] if c < 16 else v1[c - 16]
                xb = jnp.full((16,), xs, dtype=jnp.float32)
                for s in range(D_SL):
                    accs[s] = accs[s] + xb * w_v[c, pl.ds(s * 16, 16)]
            for s in range(D_SL):
                out_v[row, pl.ds(s * 16, 16)] = jnp.maximum(accs[s], 0.0)
            return c2

        lax.fori_loop(0, K, pair_body, 0)

        pltpu.sync_copy(out_v, out_hbm.at[pl.ds(out_row0, IDX_PER_CHUNK)])
        return carry

    lax.fori_loop(0, NCHUNK, chunk_body, 0)


@jax.jit
def _sc_call(table_flat, idx, x_flat, W, b):
    mesh = plsc.VectorSubcoreMesh(core_axis_name="c", subcore_axis_name="s")
    return pl.kernel(
        _sc_body,
        mesh=mesh,
        compiler_params=pltpu.CompilerParams(use_tc_tiling_on_sc=False),
        out_type=jax.ShapeDtypeStruct((BL * NF, D), jnp.float32),
        scratch_types=[
            pltpu.VMEM((NSUB, SUB), jnp.int32),
            pltpu.VMEM((IDX_PER_CHUNK, D), jnp.float32),
            pltpu.VMEM((PAIRS_PER_W * NC_PAD,), jnp.float32),
            pltpu.VMEM((NC_FEAT, D), jnp.float32),
            pltpu.VMEM((D,), jnp.float32),
            pltpu.SemaphoreType.DMA,
        ],
    )(table_flat, idx, x_flat, W, b)


def kernel(user_info_discrete, user_info_continue, tables, W, b):
    gidx = user_info_discrete.astype(jnp.int32) + (
        jnp.arange(ND, dtype=jnp.int32) * V)
    gidx = jnp.concatenate(
        [gidx.reshape(BL, ND), jnp.zeros((BL, 1), jnp.int32)], axis=1)
    idx = gidx.reshape(IDX_ROWS, SUB)
    table_flat = tables.reshape(ND * V, D)
    x_pad = jnp.pad(user_info_continue.reshape(BL, NC_FEAT),
                    ((0, 0), (0, NC_PAD - NC_FEAT)))
    x_flat = x_pad.reshape(BL * NC_PAD)
    out = _sc_call(table_flat, idx, x_flat, W, b)
    return out.reshape(B, L, NF, D)
